# baseline (speedup 1.0000x reference)
import jax
import jax.numpy as jnp
import numpy as np
from jax import lax
from jax.experimental import pallas as pl

B = 32768
N = 1024
RBLK = 512
OBLK = 1024
TOPK = 5
SPAN = np.float32(0.0197)
NEG = np.float32(-np.inf)


def _threefry_pair_py(k1, k2, c1, c2):
    M = 0xFFFFFFFF
    rot = lambda x, d: ((x << d) | (x >> (32 - d))) & M
    ks = [k1, k2, (k1 ^ k2 ^ 0x1BD11BDA) & M]
    x0, x1 = (c1 + ks[0]) & M, (c2 + ks[1]) & M
    R = [(13, 15, 26, 6), (17, 29, 16, 24)]
    for i in range(5):
        for r in R[i % 2]:
            x0 = (x0 + x1) & M
            x1 = rot(x1, r)
            x1 ^= x0
        x0 = (x0 + ks[(i + 1) % 3]) & M
        x1 = (x1 + ks[(i + 2) % 3] + i + 1) & M
    return x0, x1


_K1, _K2 = _threefry_pair_py(0, 0, 0, 1)


def _i32(x):
    return np.int32(np.uint32(x & 0xFFFFFFFF))


def _gumbel_at(j):
    shr = lax.shift_right_logical

    def rot(x, d):
        return lax.shift_left(x, jnp.int32(d)) | shr(x, jnp.int32(32 - d))

    ks = (_i32(_K1), _i32(_K2), _i32(_K1 ^ _K2 ^ 0x1BD11BDA))
    x0 = jnp.full_like(j, ks[0])
    x1 = j + ks[1]
    R = ((13, 15, 26, 6), (17, 29, 16, 24))
    for i in range(5):
        for r in R[i % 2]:
            x0 = x0 + x1
            x1 = rot(x1, r)
            x1 = x0 ^ x1
        x0 = x0 + ks[(i + 1) % 3]
        x1 = x1 + ks[(i + 2) % 3] + jnp.int32(i + 1)
    bits = x0 ^ x1
    float_bits = shr(bits, jnp.int32(9)) | jnp.int32(0x3F800000)
    f = lax.bitcast_convert_type(float_bits, jnp.float32) - jnp.float32(1.0)
    u = jnp.maximum(jnp.float32(1e-9),
                    f * (jnp.float32(1.0) - jnp.float32(1e-9))
                    + jnp.float32(1e-9))
    return jnp.float32(-0.001) * jnp.log(-jnp.log(u))


def _main_body(w_ref, flags_ref, cnt_ref, mz_ref):
    i = pl.program_id(0)
    w = w_ref[...]
    col = lax.broadcasted_iota(jnp.int32, (RBLK, N), 1)

    rcolf = jnp.float32(N - 1) - col.astype(jnp.float32)

    vals, cols = [], []
    wk = w
    for _ in range(TOPK):
        m = jnp.max(wk, axis=1, keepdims=True)
        enc = jnp.max(jnp.where(wk == m, rcolf, NEG), axis=1, keepdims=True)
        c = jnp.int32(N - 1) - enc.astype(jnp.int32)
        vals.append(m)
        cols.append(c)
        wk = jnp.where(col == c, NEG, wk)

    m0 = vals[0]
    thresh = m0 - SPAN
    resid = jnp.max(wk, axis=1, keepdims=True)
    blk_maxcnt = jnp.max((resid >= thresh).astype(jnp.int32)).reshape(1, 1)

    vm = jnp.concatenate(vals, axis=1)
    cm = jnp.concatenate(cols, axis=1)
    rowbase = (lax.broadcasted_iota(jnp.int32, (RBLK, 1), 0)
               + i * RBLK) * N
    g = _gumbel_at(rowbase + cm)
    t = vm + g
    valid = vm >= thresh
    mt = jnp.max(jnp.where(valid, t, NEG), axis=1, keepdims=True)
    best_c = jnp.min(jnp.where(valid & (t == mt), cm, N),
                     axis=1, keepdims=True)

    hit = jnp.max((col == best_c).astype(jnp.float32), axis=0, keepdims=True)
    mz_ref[...] = jnp.zeros_like(mz_ref)

    @pl.when(i == 0)
    def _init():
        flags_ref[...] = hit
        cnt_ref[...] = blk_maxcnt

    @pl.when(i != 0)
    def _acc():
        flags_ref[...] = jnp.maximum(flags_ref[...], hit)
        cnt_ref[...] = jnp.maximum(cnt_ref[...], blk_maxcnt)


def _dense_body(w_ref, g_ref, flags_ref):
    i = pl.program_id(0)
    w = w_ref[...] + g_ref[...]
    rowmax = jnp.max(w, axis=1, keepdims=True)
    col = lax.broadcasted_iota(jnp.int32, (RBLK, N), 1)
    idx = jnp.min(jnp.where(w == rowmax, col, N), axis=1, keepdims=True)
    hit = jnp.max((col == idx).astype(jnp.float32), axis=0, keepdims=True)

    @pl.when(i == 0)
    def _init():
        flags_ref[...] = hit

    @pl.when(i != 0)
    def _acc():
        flags_ref[...] = jnp.maximum(flags_ref[...], hit)


def _bcast_body(flags_ref, mz_ref, out_ref):
    del mz_ref
    out_ref[...] = jnp.broadcast_to(flags_ref[...], (OBLK, N))


def kernel(weights):
    flags_fast, maxcnt, mask_zeros = pl.pallas_call(
        _main_body,
        grid=(B // RBLK,),
        in_specs=[pl.BlockSpec((RBLK, N), lambda i: (i, 0))],
        out_specs=[
            pl.BlockSpec((1, N), lambda i: (0, 0)),
            pl.BlockSpec((1, 1), lambda i: (0, 0)),
            pl.BlockSpec((RBLK, N), lambda i: (i, 0)),
        ],
        out_shape=[
            jax.ShapeDtypeStruct((1, N), jnp.float32),
            jax.ShapeDtypeStruct((1, 1), jnp.int32),
            jax.ShapeDtypeStruct((B, N), jnp.float32),
        ],
    )(weights)

    def _fast(_):
        return flags_fast

    def _dense(_):
        noise_key = jax.random.fold_in(jax.random.key(0), 1)
        u = jax.random.uniform(noise_key, (B, N), dtype=jnp.float32,
                               minval=1e-9, maxval=1.0)
        g = jnp.float32(-0.001) * jnp.log(-jnp.log(u))
        return pl.pallas_call(
            _dense_body,
            grid=(B // RBLK,),
            in_specs=[
                pl.BlockSpec((RBLK, N), lambda i: (i, 0)),
                pl.BlockSpec((RBLK, N), lambda i: (i, 0)),
            ],
            out_specs=pl.BlockSpec((1, N), lambda i: (0, 0)),
            out_shape=jax.ShapeDtypeStruct((1, N), jnp.float32),
        )(weights, g)

    flags = lax.cond(maxcnt[0, 0] == 0, _fast, _dense, weights)

    mask = pl.pallas_call(
        _bcast_body,
        grid=(1,),
        in_specs=[
            pl.BlockSpec((N, 1), lambda i: (0, 0)),
            pl.BlockSpec((OBLK, N), lambda i: (0, 0)),
        ],
        out_specs=pl.BlockSpec((OBLK, N), lambda i: (0, 0)),
        out_shape=jax.ShapeDtypeStruct((B, N), jnp.float32),
        input_output_aliases={1: 0},
    )(flags.reshape(N, 1), mask_zeros)

    return mask


# speedup vs baseline: 1.0017x; 1.0017x over previous
import jax
import jax.numpy as jnp
import numpy as np
from jax import lax
from jax.experimental import pallas as pl

B = 32768
N = 1024
RBLK = 1024
OBLK = 1024
TOPK = 5
SPAN = np.float32(0.0197)
NEG = np.float32(-np.inf)


def _threefry_pair_py(k1, k2, c1, c2):
    M = 0xFFFFFFFF
    rot = lambda x, d: ((x << d) | (x >> (32 - d))) & M
    ks = [k1, k2, (k1 ^ k2 ^ 0x1BD11BDA) & M]
    x0, x1 = (c1 + ks[0]) & M, (c2 + ks[1]) & M
    R = [(13, 15, 26, 6), (17, 29, 16, 24)]
    for i in range(5):
        for r in R[i % 2]:
            x0 = (x0 + x1) & M
            x1 = rot(x1, r)
            x1 ^= x0
        x0 = (x0 + ks[(i + 1) % 3]) & M
        x1 = (x1 + ks[(i + 2) % 3] + i + 1) & M
    return x0, x1


_K1, _K2 = _threefry_pair_py(0, 0, 0, 1)


def _i32(x):
    return np.int32(np.uint32(x & 0xFFFFFFFF))


def _gumbel_at(j):
    shr = lax.shift_right_logical

    def rot(x, d):
        return lax.shift_left(x, jnp.int32(d)) | shr(x, jnp.int32(32 - d))

    ks = (_i32(_K1), _i32(_K2), _i32(_K1 ^ _K2 ^ 0x1BD11BDA))
    x0 = jnp.full_like(j, ks[0])
    x1 = j + ks[1]
    R = ((13, 15, 26, 6), (17, 29, 16, 24))
    for i in range(5):
        for r in R[i % 2]:
            x0 = x0 + x1
            x1 = rot(x1, r)
            x1 = x0 ^ x1
        x0 = x0 + ks[(i + 1) % 3]
        x1 = x1 + ks[(i + 2) % 3] + jnp.int32(i + 1)
    bits = x0 ^ x1
    float_bits = shr(bits, jnp.int32(9)) | jnp.int32(0x3F800000)
    f = lax.bitcast_convert_type(float_bits, jnp.float32) - jnp.float32(1.0)
    u = jnp.maximum(jnp.float32(1e-9),
                    f * (jnp.float32(1.0) - jnp.float32(1e-9))
                    + jnp.float32(1e-9))
    return jnp.float32(-0.001) * jnp.log(-jnp.log(u))


def _main_body(w_ref, flags_ref, cnt_ref, mz_ref):
    i = pl.program_id(0)
    w = w_ref[...]
    col = lax.broadcasted_iota(jnp.int32, (RBLK, N), 1)

    rcolf = jnp.float32(N - 1) - col.astype(jnp.float32)

    vals, cols = [], []
    wk = w
    for _ in range(TOPK):
        m = jnp.max(wk, axis=1, keepdims=True)
        enc = jnp.max(jnp.where(wk == m, rcolf, NEG), axis=1, keepdims=True)
        c = jnp.int32(N - 1) - enc.astype(jnp.int32)
        vals.append(m)
        cols.append(c)
        wk = jnp.where(col == c, NEG, wk)

    m0 = vals[0]
    thresh = m0 - SPAN
    count = jnp.sum((w >= thresh).astype(jnp.int32), axis=1, keepdims=True)
    blk_maxcnt = jnp.max(count).reshape(1, 1)

    vm = jnp.concatenate(vals, axis=1)
    cm = jnp.concatenate(cols, axis=1)
    rowbase = (lax.broadcasted_iota(jnp.int32, (RBLK, 1), 0)
               + i * RBLK) * N
    g = _gumbel_at(rowbase + cm)
    t = vm + g
    valid = vm >= thresh
    mt = jnp.max(jnp.where(valid, t, NEG), axis=1, keepdims=True)
    best_c = jnp.min(jnp.where(valid & (t == mt), cm, N),
                     axis=1, keepdims=True)

    hit = jnp.max((col == best_c).astype(jnp.float32), axis=0, keepdims=True)
    mz_ref[...] = jnp.zeros_like(mz_ref)

    @pl.when(i == 0)
    def _init():
        flags_ref[...] = hit
        cnt_ref[...] = blk_maxcnt

    @pl.when(i != 0)
    def _acc():
        flags_ref[...] = jnp.maximum(flags_ref[...], hit)
        cnt_ref[...] = jnp.maximum(cnt_ref[...], blk_maxcnt)


def _dense_body(w_ref, g_ref, flags_ref):
    i = pl.program_id(0)
    w = w_ref[...] + g_ref[...]
    rowmax = jnp.max(w, axis=1, keepdims=True)
    col = lax.broadcasted_iota(jnp.int32, (RBLK, N), 1)
    idx = jnp.min(jnp.where(w == rowmax, col, N), axis=1, keepdims=True)
    hit = jnp.max((col == idx).astype(jnp.float32), axis=0, keepdims=True)

    @pl.when(i == 0)
    def _init():
        flags_ref[...] = hit

    @pl.when(i != 0)
    def _acc():
        flags_ref[...] = jnp.maximum(flags_ref[...], hit)


def _bcast_body(flags_ref, mz_ref, out_ref):
    del mz_ref
    out_ref[...] = jnp.broadcast_to(flags_ref[...], (OBLK, N))


def kernel(weights):
    flags_fast, maxcnt, mask_zeros = pl.pallas_call(
        _main_body,
        grid=(B // RBLK,),
        in_specs=[pl.BlockSpec((RBLK, N), lambda i: (i, 0))],
        out_specs=[
            pl.BlockSpec((1, N), lambda i: (0, 0)),
            pl.BlockSpec((1, 1), lambda i: (0, 0)),
            pl.BlockSpec((RBLK, N), lambda i: (i, 0)),
        ],
        out_shape=[
            jax.ShapeDtypeStruct((1, N), jnp.float32),
            jax.ShapeDtypeStruct((1, 1), jnp.int32),
            jax.ShapeDtypeStruct((B, N), jnp.float32),
        ],
    )(weights)

    def _fast(_):
        return flags_fast

    def _dense(_):
        noise_key = jax.random.fold_in(jax.random.key(0), 1)
        u = jax.random.uniform(noise_key, (B, N), dtype=jnp.float32,
                               minval=1e-9, maxval=1.0)
        g = jnp.float32(-0.001) * jnp.log(-jnp.log(u))
        return pl.pallas_call(
            _dense_body,
            grid=(B // RBLK,),
            in_specs=[
                pl.BlockSpec((RBLK, N), lambda i: (i, 0)),
                pl.BlockSpec((RBLK, N), lambda i: (i, 0)),
            ],
            out_specs=pl.BlockSpec((1, N), lambda i: (0, 0)),
            out_shape=jax.ShapeDtypeStruct((1, N), jnp.float32),
        )(weights, g)

    flags = lax.cond(maxcnt[0, 0] <= TOPK, _fast, _dense, weights)

    mask = pl.pallas_call(
        _bcast_body,
        grid=(1,),
        in_specs=[
            pl.BlockSpec((N, 1), lambda i: (0, 0)),
            pl.BlockSpec((OBLK, N), lambda i: (0, 0)),
        ],
        out_specs=pl.BlockSpec((OBLK, N), lambda i: (0, 0)),
        out_shape=jax.ShapeDtypeStruct((B, N), jnp.float32),
        input_output_aliases={1: 0},
    )(flags.reshape(N, 1), mask_zeros)

    return mask


# speedup vs baseline: 1.0058x; 1.0040x over previous
import jax
import jax.numpy as jnp
import numpy as np
from jax import lax
from jax.experimental import pallas as pl

B = 32768
N = 1024
RBLK = 512
OBLK = 1024
TOPK = 5
SPAN = np.float32(0.0197)
NEG = np.float32(-np.inf)


def _threefry_pair_py(k1, k2, c1, c2):
    M = 0xFFFFFFFF
    rot = lambda x, d: ((x << d) | (x >> (32 - d))) & M
    ks = [k1, k2, (k1 ^ k2 ^ 0x1BD11BDA) & M]
    x0, x1 = (c1 + ks[0]) & M, (c2 + ks[1]) & M
    R = [(13, 15, 26, 6), (17, 29, 16, 24)]
    for i in range(5):
        for r in R[i % 2]:
            x0 = (x0 + x1) & M
            x1 = rot(x1, r)
            x1 ^= x0
        x0 = (x0 + ks[(i + 1) % 3]) & M
        x1 = (x1 + ks[(i + 2) % 3] + i + 1) & M
    return x0, x1


_K1, _K2 = _threefry_pair_py(0, 0, 0, 1)


def _i32(x):
    return np.int32(np.uint32(x & 0xFFFFFFFF))


def _gumbel_at(j):
    shr = lax.shift_right_logical

    def rot(x, d):
        return lax.shift_left(x, jnp.int32(d)) | shr(x, jnp.int32(32 - d))

    ks = (_i32(_K1), _i32(_K2), _i32(_K1 ^ _K2 ^ 0x1BD11BDA))
    x0 = jnp.full_like(j, ks[0])
    x1 = j + ks[1]
    R = ((13, 15, 26, 6), (17, 29, 16, 24))
    for i in range(5):
        for r in R[i % 2]:
            x0 = x0 + x1
            x1 = rot(x1, r)
            x1 = x0 ^ x1
        x0 = x0 + ks[(i + 1) % 3]
        x1 = x1 + ks[(i + 2) % 3] + jnp.int32(i + 1)
    bits = x0 ^ x1
    float_bits = shr(bits, jnp.int32(9)) | jnp.int32(0x3F800000)
    f = lax.bitcast_convert_type(float_bits, jnp.float32) - jnp.float32(1.0)
    u = jnp.maximum(jnp.float32(1e-9),
                    f * (jnp.float32(1.0) - jnp.float32(1e-9))
                    + jnp.float32(1e-9))
    return jnp.float32(-0.001) * jnp.log(-jnp.log(u))


def _main_body(w_ref, flags_ref, cnt_ref, mz_ref):
    i = pl.program_id(0)
    w = w_ref[...]
    col = lax.broadcasted_iota(jnp.int32, (RBLK, N), 1)

    rcolf = jnp.float32(N - 1) - col.astype(jnp.float32)

    vals, cols = [], []
    wk = w
    for _ in range(TOPK):
        m = jnp.max(wk, axis=1, keepdims=True)
        enc = jnp.max(jnp.where(wk == m, rcolf, NEG), axis=1, keepdims=True)
        c = jnp.int32(N - 1) - enc.astype(jnp.int32)
        vals.append(m)
        cols.append(c)
        wk = jnp.where(col == c, NEG, wk)

    m0 = vals[0]
    thresh = m0 - SPAN
    count = jnp.sum((w >= thresh).astype(jnp.int32), axis=1, keepdims=True)
    blk_maxcnt = jnp.max(count).reshape(1, 1)

    vm = jnp.concatenate(vals, axis=1)
    cm = jnp.concatenate(cols, axis=1)
    rowbase = (lax.broadcasted_iota(jnp.int32, (RBLK, 1), 0)
               + i * RBLK) * N
    g = _gumbel_at(rowbase + cm)
    t = vm + g
    valid = vm >= thresh
    mt = jnp.max(jnp.where(valid, t, NEG), axis=1, keepdims=True)
    best_c = jnp.min(jnp.where(valid & (t == mt), cm, N),
                     axis=1, keepdims=True)

    hit = jnp.max((col == best_c).astype(jnp.float32), axis=0, keepdims=True)
    mz_ref[...] = jnp.zeros_like(mz_ref)

    @pl.when(i == 0)
    def _init():
        flags_ref[...] = hit
        cnt_ref[...] = blk_maxcnt

    @pl.when(i != 0)
    def _acc():
        flags_ref[...] = jnp.maximum(flags_ref[...], hit)
        cnt_ref[...] = jnp.maximum(cnt_ref[...], blk_maxcnt)


def _dense_body(w_ref, g_ref, flags_ref):
    i = pl.program_id(0)
    w = w_ref[...] + g_ref[...]
    rowmax = jnp.max(w, axis=1, keepdims=True)
    col = lax.broadcasted_iota(jnp.int32, (RBLK, N), 1)
    idx = jnp.min(jnp.where(w == rowmax, col, N), axis=1, keepdims=True)
    hit = jnp.max((col == idx).astype(jnp.float32), axis=0, keepdims=True)

    @pl.when(i == 0)
    def _init():
        flags_ref[...] = hit

    @pl.when(i != 0)
    def _acc():
        flags_ref[...] = jnp.maximum(flags_ref[...], hit)


def _bcast_body(flags_ref, mz_ref, out_ref):
    del mz_ref
    out_ref[...] = jnp.broadcast_to(flags_ref[...], (OBLK, N))


def kernel(weights):
    flags_fast, maxcnt, mask_zeros = pl.pallas_call(
        _main_body,
        grid=(B // RBLK,),
        in_specs=[pl.BlockSpec((RBLK, N), lambda i: (i, 0))],
        out_specs=[
            pl.BlockSpec((1, N), lambda i: (0, 0)),
            pl.BlockSpec((1, 1), lambda i: (0, 0)),
            pl.BlockSpec((RBLK, N), lambda i: (i, 0)),
        ],
        out_shape=[
            jax.ShapeDtypeStruct((1, N), jnp.float32),
            jax.ShapeDtypeStruct((1, 1), jnp.int32),
            jax.ShapeDtypeStruct((B, N), jnp.float32),
        ],
    )(weights)

    def _fast(_):
        return flags_fast

    def _dense(_):
        noise_key = jax.random.fold_in(jax.random.key(0), 1)
        u = jax.random.uniform(noise_key, (B, N), dtype=jnp.float32,
                               minval=1e-9, maxval=1.0)
        g = jnp.float32(-0.001) * jnp.log(-jnp.log(u))
        return pl.pallas_call(
            _dense_body,
            grid=(B // RBLK,),
            in_specs=[
                pl.BlockSpec((RBLK, N), lambda i: (i, 0)),
                pl.BlockSpec((RBLK, N), lambda i: (i, 0)),
            ],
            out_specs=pl.BlockSpec((1, N), lambda i: (0, 0)),
            out_shape=jax.ShapeDtypeStruct((1, N), jnp.float32),
        )(weights, g)

    flags = lax.cond(maxcnt[0, 0] <= TOPK, _fast, _dense, weights)

    mask = pl.pallas_call(
        _bcast_body,
        grid=(1,),
        in_specs=[
            pl.BlockSpec((N, 1), lambda i: (0, 0)),
            pl.BlockSpec((OBLK, N), lambda i: (0, 0)),
        ],
        out_specs=pl.BlockSpec((OBLK, N), lambda i: (0, 0)),
        out_shape=jax.ShapeDtypeStruct((B, N), jnp.float32),
        input_output_aliases={1: 0},
    )(flags.reshape(N, 1), mask_zeros)

    return mask
